# baseline (speedup 1.0000x reference)
import functools

import jax
import jax.numpy as jnp
from jax import lax
from jax.experimental import pallas as pl
from jax.experimental.pallas import tpu as pltpu
from jax.experimental.pallas import tpu_sc as plsc

N = 10000
E = 160000
D = 256
HOPS = 6
NC = 2
NS = 16
NW = NC * NS
HF = 128
RPT = N // NS
C = 64
NCH = 160
EPT = NCH * C
PAD = NS * EPT - E
SB = 25

_mesh = plsc.VectorSubcoreMesh(core_axis_name="c", subcore_axis_name="s")
_UNTILED = pltpu.CompilerParams(use_tc_tiling_on_sc=False)


@functools.partial(
    pl.kernel,
    mesh=_mesh,
    out_type=jax.ShapeDtypeStruct((NC, N + 8, 16), jnp.float32),
    scratch_types=[
        pltpu.VMEM((NCH, C), jnp.int32),
        pltpu.VMEM((C, 16), jnp.float32),
        pltpu.VMEM_SHARED((N + 8, 16), jnp.float32),
    ],
    compiler_params=_UNTILED,
)
def _deg_kernel(dst4_hbm, ones_hbm, zeros_hbm, degp_hbm, dst_v, ones_v, acc_sh):
    cid = lax.axis_index("c")
    sid = lax.axis_index("s")
    pltpu.sync_copy(dst4_hbm.at[cid, sid], dst_v)
    pltpu.sync_copy(ones_hbm, ones_v)

    @pl.when(sid == 0)
    def _():
        pltpu.sync_copy(zeros_hbm, acc_sh)

    plsc.subcore_barrier()

    def chunk(cc, carry):
        pltpu.sync_copy(ones_v, acc_sh.at[dst_v.at[cc]], add=True)
        return carry

    lax.fori_loop(0, NCH, chunk, 0)
    plsc.subcore_barrier()

    @pl.when(sid == 0)
    def _():
        pltpu.sync_copy(acc_sh, degp_hbm.at[cid])


_BM1 = 1000


def _in_tf_body(x_ref, w_ref, b_ref, dp_ref, g0_ref, invb_ref, sq_ref):
    fh = pl.program_id(1)
    z = jnp.dot(x_ref[...], w_ref[...], preferred_element_type=jnp.float32)
    z = jnp.maximum(z + b_ref[...], 0.0)
    deg = dp_ref[:, 0] + 1.0
    dinv = lax.rsqrt(deg)
    half = jnp.where(fh == 0, z[:, :HF], z[:, HF:])
    g0_ref[...] = half * dinv[:, None]
    invb_ref[...] = jnp.broadcast_to((1.0 / deg)[:, None], (_BM1, HF))
    sq_ref[...] = jnp.sqrt(deg)[:, None]


def _in_tf(x, w, b, dp):
    nm = N // _BM1
    return pl.pallas_call(
        _in_tf_body,
        grid=(nm, 2),
        in_specs=[
            pl.BlockSpec((_BM1, D), lambda m, f: (m, 0)),
            pl.BlockSpec((D, D), lambda m, f: (0, 0)),
            pl.BlockSpec((1, D), lambda m, f: (0, 0)),
            pl.BlockSpec((_BM1, 16), lambda m, f: (m, 0)),
        ],
        out_specs=[
            pl.BlockSpec((_BM1, HF), lambda m, f: (f * (N // _BM1) + m, 0)),
            pl.BlockSpec((_BM1, HF), lambda m, f: (m, 0)),
            pl.BlockSpec((_BM1, 1), lambda m, f: (m, 0)),
        ],
        out_shape=[
            jax.ShapeDtypeStruct((NC * N, HF), jnp.float32),
            jax.ShapeDtypeStruct((N, HF), jnp.float32),
            jax.ShapeDtypeStruct((N, 1), jnp.float32),
        ],
    )(x, w, b, dp)


HW = HF // 2


@functools.partial(
    pl.kernel,
    mesh=_mesh,
    out_type=[
        jax.ShapeDtypeStruct((HOPS + 1, NC * N, HF), jnp.float32),
        jax.ShapeDtypeStruct((HOPS, NC * N, HF // 2), jnp.int32),
    ],
    scratch_types=[
        pltpu.VMEM((NCH, C), jnp.int32),
        pltpu.VMEM((NCH, C), jnp.int32),
        pltpu.VMEM((C, HF // 2), jnp.int32),
        pltpu.VMEM((C, HF // 2), jnp.int32),
        pltpu.VMEM((C, HF), jnp.float32),
        pltpu.VMEM((C, HF), jnp.float32),
        pltpu.VMEM((SB, HF // 2), jnp.int32),
        pltpu.VMEM_SHARED((N + 8, HF), jnp.float32),
        pltpu.SemaphoreType.DMA,
        pltpu.SemaphoreType.DMA,
        pltpu.SemaphoreType.DMA,
        pltpu.SemaphoreType.DMA,
    ],
    compiler_params=_UNTILED,
)
def _hops_kernel(g0_hbm, g0p_hbm, src_hbm, dst4_hbm, invb_hbm,
                 gall_hbm, gb_hbm,
                 src_v, dst_v, bfa_v, bfb_v, fa_v, fb_v, pk_v,
                 acc_sh, sga, sgb, ssa, ssb):
    cid = lax.axis_index("c")
    sid = lax.axis_index("s")
    tb = cid * N
    row0 = sid * RPT

    pltpu.sync_copy(src_hbm.at[cid, sid], src_v)
    pltpu.sync_copy(dst4_hbm.at[cid, sid], dst_v)
    pltpu.sync_copy(g0_hbm.at[pl.ds(tb + row0, RPT)],
                    gall_hbm.at[0, pl.ds(tb + row0, RPT)])
    pltpu.sync_copy(g0_hbm.at[pl.ds(tb + row0, RPT)],
                    acc_sh.at[pl.ds(row0, RPT)])
    pltpu.sync_copy(g0p_hbm.at[pl.ds(tb + row0, RPT)],
                    gb_hbm.at[0, pl.ds(tb + row0, RPT)])
    plsc.subcore_barrier()

    bufs = [(bfa_v, fa_v, sga, ssa), (bfb_v, fb_v, sgb, ssb)]

    for k in range(HOPS):

        def issue(c, off):
            bf, _, sg, _ = bufs[off]
            pltpu.async_copy(gb_hbm.at[k].at[src_v.at[c]], bf, sg)

        def slot(c, off, first, last):
            bf, fv, sg, ss = bufs[off]
            if not first:
                pltpu.make_async_copy(fv, acc_sh.at[dst_v.at[c - 2]], ss).wait()
            pltpu.make_async_copy(gb_hbm.at[k].at[src_v.at[c]], bf, sg).wait()

            def urow(r, carry):
                for w in range(HW // 16):
                    wv = bf[r, pl.ds(w * 16, 16)]
                    lo = lax.bitcast_convert_type(
                        lax.shift_left(wv, 16), jnp.float32)
                    hi = lax.bitcast_convert_type(
                        wv & jnp.int32(-65536), jnp.float32)
                    fv[r, pl.ds(w * 32, 16)] = lo
                    fv[r, pl.ds(w * 32 + 16, 16)] = hi
                return carry

            lax.fori_loop(0, C, urow, 0)
            pltpu.async_copy(fv, acc_sh.at[dst_v.at[c]], ss, add=True)
            if not last:
                issue(c + 2, off)

        issue(0, 0)
        issue(1, 1)
        slot(0, 0, True, False)
        slot(1, 1, True, False)

        def pairbody(i, carry):
            c0 = 2 * i
            slot(c0, 0, False, False)
            slot(c0 + 1, 1, False, False)
            return carry

        lax.fori_loop(1, NCH // 2 - 1, pairbody, 0)
        slot(NCH - 2, 0, False, True)
        slot(NCH - 1, 1, False, True)
        pltpu.make_async_copy(fa_v, acc_sh.at[dst_v.at[NCH - 2]], ssa).wait()
        pltpu.make_async_copy(fb_v, acc_sh.at[dst_v.at[NCH - 1]], ssb).wait()
        plsc.subcore_barrier()

        def wb(t, carry):
            r0 = row0 + t * SB
            pltpu.sync_copy(acc_sh.at[pl.ds(r0, SB)], fb_v.at[pl.ds(0, SB)])
            pltpu.sync_copy(invb_hbm.at[pl.ds(r0, SB)], fa_v.at[pl.ds(0, SB)])

            def srow(r, carry2):
                for j in range(HF // 16):
                    sl = pl.ds(j * 16, 16)
                    fb_v[r, sl] = fb_v[r, sl] * fa_v[r, sl]
                for w in range(HW // 16):
                    x = lax.bitcast_convert_type(
                        fb_v[r, pl.ds(w * 32, 16)], jnp.int32)
                    y = lax.bitcast_convert_type(
                        fb_v[r, pl.ds(w * 32 + 16, 16)], jnp.int32)
                    word = ((y + 32768) & jnp.int32(-65536)) | (
                        lax.shift_right_logical(x + 32768, 16))
                    pk_v[r, pl.ds(w * 16, 16)] = word
                return carry2

            lax.fori_loop(0, SB, srow, 0)
            pltpu.sync_copy(fb_v.at[pl.ds(0, SB)], gall_hbm.at[k + 1, pl.ds(tb + r0, SB)])
            pltpu.sync_copy(fb_v.at[pl.ds(0, SB)], acc_sh.at[pl.ds(r0, SB)])
            if k + 1 < HOPS:
                pltpu.sync_copy(pk_v, gb_hbm.at[k + 1, pl.ds(tb + r0, SB)])
            return carry

        lax.fori_loop(0, RPT // SB, wb, 0)
        plsc.subcore_barrier()


_BM2 = 1000


def _comb_body(g_ref, w_ref, sq_ref, b_ref, gam_ref, bet_ref, o_ref, acc):
    kk = pl.program_id(1)

    @pl.when(kk == 0)
    def _():
        acc[...] = jnp.zeros_like(acc)

    acc[...] += jnp.dot(g_ref[0], w_ref[0, 0], preferred_element_type=jnp.float32)

    @pl.when(kk == 2 * (HOPS + 1) - 1)
    def _():
        y = acc[...] * sq_ref[...] + b_ref[...]
        y = jnp.maximum(y, 0.0)
        mu = jnp.mean(y, axis=1, keepdims=True)
        var = jnp.mean((y - mu) ** 2, axis=1, keepdims=True)
        o_ref[...] = (y - mu) * lax.rsqrt(var + 1e-5) * gam_ref[...] + bet_ref[...]


def _comb(gall, w4, sq, b, gam, bet):
    nm = N // _BM2
    nk = 2 * (HOPS + 1)
    return pl.pallas_call(
        _comb_body,
        grid=(nm, nk),
        in_specs=[
            pl.BlockSpec((1, _BM2, HF), lambda m, kk: (kk // 2, (kk % 2) * nm + m, 0)),
            pl.BlockSpec((1, 1, HF, D), lambda m, kk: (kk // 2, kk % 2, 0, 0)),
            pl.BlockSpec((_BM2, 1), lambda m, kk: (m, 0)),
            pl.BlockSpec((1, D), lambda m, kk: (0, 0)),
            pl.BlockSpec((1, D), lambda m, kk: (0, 0)),
            pl.BlockSpec((1, D), lambda m, kk: (0, 0)),
        ],
        out_specs=pl.BlockSpec((_BM2, D), lambda m, kk: (m, 0)),
        out_shape=jax.ShapeDtypeStruct((N, D), jnp.float32),
        scratch_shapes=[pltpu.VMEM((_BM2, D), jnp.float32)],
    )(gall, w4, sq, b, gam, bet)


def kernel(features, edge_index, W_in, b_in, W_fc, b_fc, gamma, beta):
    src = edge_index[0]
    dst = edge_index[1]
    srcp = jnp.concatenate([src, jnp.zeros((PAD,), jnp.int32)])
    dstp = jnp.concatenate([dst, jnp.full((PAD,), N, jnp.int32)])
    half_off = (jnp.arange(NC, dtype=jnp.int32) * N)[:, None, None, None]
    src3 = srcp.reshape(1, NS, NCH, C) + half_off
    dst4 = jnp.broadcast_to(dstp.reshape(1, NS, NCH, C), (NC, NS, NCH, C))
    ones = jnp.ones((C, 16), jnp.float32)
    zer = jnp.zeros((N + 8, 16), jnp.float32)

    degp = _deg_kernel(dst4, ones, zer)
    g0, invb, sq_d = _in_tf(features, W_in, b_in.reshape(1, D), degp[0, :N])
    gi = lax.bitcast_convert_type(g0.reshape(NC * N, 4, 2, 16), jnp.int32)
    gi = gi + 32768
    g0p = ((gi[:, :, 1, :] & jnp.int32(-65536))
           | lax.shift_right_logical(gi[:, :, 0, :], 16)).reshape(NC * N, HF // 2)
    gall, _ = _hops_kernel(g0, g0p, src3, dst4, invb)
    w4 = W_fc.reshape(HOPS + 1, NC, HF, D)
    return _comb(gall, w4, sq_d, b_fc.reshape(1, D), gamma.reshape(1, D),
                 beta.reshape(1, D))


# speedup vs baseline: 1.5235x; 1.5235x over previous
import functools

import jax
import jax.numpy as jnp
from jax import lax
from jax.experimental import pallas as pl
from jax.experimental.pallas import tpu as pltpu
from jax.experimental.pallas import tpu_sc as plsc

N = 10000
E = 160000
D = 256
HOPS = 6
NC = 2
NS = 16
NW = NC * NS
HF = 128
RPT = N // NS
C = 80
NCH = 126
EPT = NCH * C
PAD = NS * EPT - E
SB = 25

_mesh = plsc.VectorSubcoreMesh(core_axis_name="c", subcore_axis_name="s")
_UNTILED = pltpu.CompilerParams(use_tc_tiling_on_sc=False)


@functools.partial(
    pl.kernel,
    mesh=_mesh,
    out_type=jax.ShapeDtypeStruct((NC, N + 8, 16), jnp.float32),
    scratch_types=[
        pltpu.VMEM((NCH, C), jnp.int32),
        pltpu.VMEM((C, 16), jnp.float32),
        pltpu.VMEM_SHARED((N + 8, 16), jnp.float32),
    ],
    compiler_params=_UNTILED,
)
def _deg_kernel(dst4_hbm, ones_hbm, zeros_hbm, degp_hbm, dst_v, ones_v, acc_sh):
    cid = lax.axis_index("c")
    sid = lax.axis_index("s")
    pltpu.sync_copy(dst4_hbm.at[cid, sid], dst_v)
    pltpu.sync_copy(ones_hbm, ones_v)

    @pl.when(sid == 0)
    def _():
        pltpu.sync_copy(zeros_hbm, acc_sh)

    plsc.subcore_barrier()

    def chunk(cc, carry):
        pltpu.sync_copy(ones_v, acc_sh.at[dst_v.at[cc]], add=True)
        return carry

    lax.fori_loop(0, NCH, chunk, 0)
    plsc.subcore_barrier()

    @pl.when(sid == 0)
    def _():
        pltpu.sync_copy(acc_sh, degp_hbm.at[cid])


_BM1 = 1000


def _in_tf_body(x_ref, w_ref, b_ref, dp_ref, g0_ref, invb_ref, sq_ref):
    fh = pl.program_id(1)
    z = jnp.dot(x_ref[...], w_ref[...], preferred_element_type=jnp.float32)
    z = jnp.maximum(z + b_ref[...], 0.0)
    deg = dp_ref[:, 0] + 1.0
    dinv = lax.rsqrt(deg)
    half = jnp.where(fh == 0, z[:, :HF], z[:, HF:])
    g0_ref[...] = half * dinv[:, None]
    invb_ref[...] = jnp.broadcast_to((1.0 / deg)[:, None], (_BM1, HF))
    sq_ref[...] = jnp.sqrt(deg)[:, None]


def _in_tf(x, w, b, dp):
    nm = N // _BM1
    return pl.pallas_call(
        _in_tf_body,
        grid=(nm, 2),
        in_specs=[
            pl.BlockSpec((_BM1, D), lambda m, f: (m, 0)),
            pl.BlockSpec((D, D), lambda m, f: (0, 0)),
            pl.BlockSpec((1, D), lambda m, f: (0, 0)),
            pl.BlockSpec((_BM1, 16), lambda m, f: (m, 0)),
        ],
        out_specs=[
            pl.BlockSpec((_BM1, HF), lambda m, f: (f * (N // _BM1) + m, 0)),
            pl.BlockSpec((_BM1, HF), lambda m, f: (m, 0)),
            pl.BlockSpec((_BM1, 1), lambda m, f: (m, 0)),
        ],
        out_shape=[
            jax.ShapeDtypeStruct((NC * N, HF), jnp.float32),
            jax.ShapeDtypeStruct((N, HF), jnp.float32),
            jax.ShapeDtypeStruct((N, 1), jnp.float32),
        ],
    )(x, w, b, dp)


@functools.partial(
    pl.kernel,
    mesh=_mesh,
    out_type=jax.ShapeDtypeStruct((HOPS + 1, NC * N, HF), jnp.float32),
    scratch_types=[
        pltpu.VMEM((NCH, C), jnp.int32),
        pltpu.VMEM((NCH, C), jnp.int32),
        pltpu.VMEM((C, HF), jnp.float32),
        pltpu.VMEM((C, HF), jnp.float32),
        pltpu.VMEM((C, HF), jnp.float32),
        pltpu.VMEM_SHARED((N + 8, HF), jnp.float32),
        pltpu.SemaphoreType.DMA,
        pltpu.SemaphoreType.DMA,
        pltpu.SemaphoreType.DMA,
    ],
    compiler_params=_UNTILED,
)
def _hops_kernel(g0_hbm, src_hbm, dst4_hbm, invb_hbm, gall_hbm,
                 src_v, dst_v, rowsa_v, rowsb_v, rowsc_v,
                 acc_sh, sema, semb, semc):
    cid = lax.axis_index("c")
    sid = lax.axis_index("s")
    tb = cid * N
    row0 = sid * RPT

    pltpu.sync_copy(src_hbm.at[cid, sid], src_v)
    pltpu.sync_copy(dst4_hbm.at[cid, sid], dst_v)
    pltpu.sync_copy(g0_hbm.at[pl.ds(tb + row0, RPT)],
                    gall_hbm.at[0, pl.ds(tb + row0, RPT)])
    pltpu.sync_copy(g0_hbm.at[pl.ds(tb + row0, RPT)],
                    acc_sh.at[pl.ds(row0, RPT)])
    plsc.subcore_barrier()

    for k in range(HOPS):

        def issue(c, rows, sem):
            pltpu.async_copy(gall_hbm.at[k].at[src_v.at[c]], rows, sem)

        def wait(c, rows, sem):
            pltpu.make_async_copy(gall_hbm.at[k].at[src_v.at[c]], rows, sem).wait()

        def scat(c, rows):
            pltpu.sync_copy(rows, acc_sh.at[dst_v.at[c]], add=True)

        issue(0, rowsa_v, sema)
        issue(1, rowsb_v, semb)

        def tri(c0, more):
            issue(c0 + 2, rowsc_v, semc)
            wait(c0, rowsa_v, sema)
            scat(c0, rowsa_v)
            if more:
                issue(c0 + 3, rowsa_v, sema)
            wait(c0 + 1, rowsb_v, semb)
            scat(c0 + 1, rowsb_v)
            if more:
                issue(c0 + 4, rowsb_v, semb)
            wait(c0 + 2, rowsc_v, semc)
            scat(c0 + 2, rowsc_v)

        def tribody(i, carry):
            tri(3 * i, True)
            return carry

        lax.fori_loop(0, NCH // 3 - 1, tribody, 0)
        tri(NCH - 3, False)
        plsc.subcore_barrier()

        def wb(t, carry):
            r0 = row0 + t * SB
            pltpu.sync_copy(acc_sh.at[pl.ds(r0, SB)], rowsb_v.at[pl.ds(0, SB)])
            pltpu.sync_copy(invb_hbm.at[pl.ds(r0, SB)], rowsa_v.at[pl.ds(0, SB)])

            def srow(r, carry2):
                for j in range(HF // 16):
                    sl = pl.ds(j * 16, 16)
                    rowsb_v[r, sl] = rowsb_v[r, sl] * rowsa_v[r, sl]
                return carry2

            lax.fori_loop(0, SB, srow, 0)
            pltpu.sync_copy(rowsb_v.at[pl.ds(0, SB)], gall_hbm.at[k + 1, pl.ds(tb + r0, SB)])
            pltpu.sync_copy(rowsb_v.at[pl.ds(0, SB)], acc_sh.at[pl.ds(r0, SB)])
            return carry

        lax.fori_loop(0, RPT // SB, wb, 0)
        plsc.subcore_barrier()


_BM2 = 1000


def _comb_body(g_ref, w_ref, sq_ref, b_ref, gam_ref, bet_ref, o_ref, acc):
    kk = pl.program_id(1)

    @pl.when(kk == 0)
    def _():
        acc[...] = jnp.zeros_like(acc)

    acc[...] += jnp.dot(g_ref[0], w_ref[0, 0], preferred_element_type=jnp.float32)

    @pl.when(kk == 2 * (HOPS + 1) - 1)
    def _():
        y = acc[...] * sq_ref[...] + b_ref[...]
        y = jnp.maximum(y, 0.0)
        mu = jnp.mean(y, axis=1, keepdims=True)
        var = jnp.mean((y - mu) ** 2, axis=1, keepdims=True)
        o_ref[...] = (y - mu) * lax.rsqrt(var + 1e-5) * gam_ref[...] + bet_ref[...]


def _comb(gall, w4, sq, b, gam, bet):
    nm = N // _BM2
    nk = 2 * (HOPS + 1)
    return pl.pallas_call(
        _comb_body,
        grid=(nm, nk),
        in_specs=[
            pl.BlockSpec((1, _BM2, HF), lambda m, kk: (kk // 2, (kk % 2) * nm + m, 0)),
            pl.BlockSpec((1, 1, HF, D), lambda m, kk: (kk // 2, kk % 2, 0, 0)),
            pl.BlockSpec((_BM2, 1), lambda m, kk: (m, 0)),
            pl.BlockSpec((1, D), lambda m, kk: (0, 0)),
            pl.BlockSpec((1, D), lambda m, kk: (0, 0)),
            pl.BlockSpec((1, D), lambda m, kk: (0, 0)),
        ],
        out_specs=pl.BlockSpec((_BM2, D), lambda m, kk: (m, 0)),
        out_shape=jax.ShapeDtypeStruct((N, D), jnp.float32),
        scratch_shapes=[pltpu.VMEM((_BM2, D), jnp.float32)],
    )(gall, w4, sq, b, gam, bet)


def kernel(features, edge_index, W_in, b_in, W_fc, b_fc, gamma, beta):
    src = edge_index[0]
    dst = edge_index[1]
    srcp = jnp.concatenate([src, jnp.zeros((PAD,), jnp.int32)])
    dstp = jnp.concatenate([dst, jnp.full((PAD,), N, jnp.int32)])
    half_off = (jnp.arange(NC, dtype=jnp.int32) * N)[:, None, None, None]
    src3 = srcp.reshape(1, NS, NCH, C) + half_off
    dst4 = jnp.broadcast_to(dstp.reshape(1, NS, NCH, C), (NC, NS, NCH, C))
    ones = jnp.ones((C, 16), jnp.float32)
    zer = jnp.zeros((N + 8, 16), jnp.float32)

    degp = _deg_kernel(dst4, ones, zer)
    g0, invb, sq_d = _in_tf(features, W_in, b_in.reshape(1, D), degp[0, :N])
    gall = _hops_kernel(g0, src3, dst4, invb)
    w4 = W_fc.reshape(HOPS + 1, NC, HF, D)
    return _comb(gall, w4, sq_d, b_fc.reshape(1, D), gamma.reshape(1, D),
                 beta.reshape(1, D))
